# baseline (device time: 59427 ns/iter reference)
import jax
import jax.numpy as jnp
from jax import lax
from jax.experimental import pallas as pl
from jax.experimental.pallas import tpu as pltpu

N_DEV = 8
B = 2
SQ = 512
SKV = 512
H_PER = 8
DH = 64
D_MODEL = 768
D_HID = H_PER * DH

PCOL = D_MODEL // 3

PIECES = ((0, 0, SQ), (1, 0, SQ // 2), (1, SQ // 2, SQ // 2))

ORDER = ((2, 1, 0), (1, 0, 2), (0, 2, 1))


def kernel(x, Wq, K_ext, V_ext, Wo):
    def body(x_ref, wq_ref, k_ref, v_ref, wo_ref, out_ref,
             send_ref, recv_ref, send_sems, recv_sems):
        my = lax.axis_index("i")
        lab = my ^ ((my >> 1) & 1)

        def pos_of(l):
            return l ^ ((l >> 1) & 1)

        qi = lax.broadcasted_iota(jnp.int32, (SQ, SKV), 0)
        ki = lax.broadcasted_iota(jnp.int32, (SQ, SKV), 1)
        mask = (jnp.abs(qi - ki) <= 128) | (ki < 32) | (qi < 32)
        neg = jnp.float32(-1e9)
        bf16 = jnp.bfloat16

        wq_s = wq_ref[:, pl.ds(my * D_HID, D_HID)] * 0.125

        def project_q(b):
            return jnp.dot(x_ref[b].astype(bf16), wq_s.astype(bf16),
                           preferred_element_type=jnp.float32)

        def attn_rows(qb, b, r0, nr):
            acc = jnp.zeros((nr, D_MODEL), jnp.float32)
            for h in range(H_PER):
                qh = qb[r0:r0 + nr, h * DH:(h + 1) * DH].astype(bf16)
                kh = k_ref[b, :, h, :].astype(bf16)
                s = lax.dot_general(
                    qh, kh, (((1,), (1,)), ((), ())),
                    preferred_element_type=jnp.float32)
                e = jnp.exp(jnp.where(mask[r0:r0 + nr, :], s, neg))
                ctx = jnp.dot(e, v_ref[b, :, h, :],
                              preferred_element_type=jnp.float32)
                ctx = ctx / jnp.sum(e, axis=1, keepdims=True)
                acc = acc + jnp.dot(
                    ctx.astype(bf16),
                    wo_ref[pl.ds(my * D_HID + h * DH, DH), :].astype(bf16),
                    preferred_element_type=jnp.float32)
            send_ref[b, r0:r0 + nr, :] = acc.astype(bf16)

        def start_round(p, r):
            b, r0, nr = PIECES[p]
            inflight = []
            for j in range(3):
                bit = ORDER[j][r]
                partner = pos_of(lab ^ (1 << bit))
                rdma = pltpu.make_async_remote_copy(
                    src_ref=send_ref.at[b, pl.ds(r0, nr),
                                        pl.ds(j * PCOL, PCOL)],
                    dst_ref=recv_ref.at[p, r, j, pl.ds(0, nr)],
                    send_sem=send_sems.at[p, r, j],
                    recv_sem=recv_sems.at[p, r, j],
                    device_id=(partner,),
                    device_id_type=pl.DeviceIdType.MESH,
                )
                rdma.start()
                inflight.append(rdma)
            return inflight

        def finish_round(p, r, inflight):
            b, r0, nr = PIECES[p]
            for rdma in inflight:
                rdma.wait()
            for j in range(3):
                rows = pl.ds(r0, nr)
                cols = pl.ds(j * PCOL, PCOL)
                send_ref[b, rows, cols] = (
                    send_ref[b, rows, cols] + recv_ref[p, r, j, :nr])
            if r == 2:
                out_ref[b, r0:r0 + nr, :] = (
                    send_ref[b, r0:r0 + nr, :].astype(jnp.float32))

        barrier = pltpu.get_barrier_semaphore()
        for bit in range(3):
            pl.semaphore_signal(
                barrier, inc=1,
                device_id=(pos_of(lab ^ (1 << bit)),),
                device_id_type=pl.DeviceIdType.MESH,
            )

        qb0 = project_q(0)
        attn_rows(qb0, 0, 0, SQ)
        pl.semaphore_wait(barrier, 3)
        fly_a = start_round(0, 0)
        qb1 = project_q(1)
        attn_rows(qb1, 1, 0, SQ // 2)
        fly_b = start_round(1, 0)
        finish_round(0, 0, fly_a)
        fly_a = start_round(0, 1)
        attn_rows(qb1, 1, SQ // 2, SQ // 2)
        fly_c = start_round(2, 0)
        finish_round(0, 1, fly_a)
        fly_a = start_round(0, 2)
        finish_round(1, 0, fly_b)
        fly_b = start_round(1, 1)
        finish_round(0, 2, fly_a)
        finish_round(2, 0, fly_c)
        fly_c = start_round(2, 1)
        finish_round(1, 1, fly_b)
        fly_b = start_round(1, 2)
        finish_round(2, 1, fly_c)
        fly_c = start_round(2, 2)
        finish_round(1, 2, fly_b)
        finish_round(2, 2, fly_c)

    return pl.pallas_call(
        body,
        out_shape=jax.ShapeDtypeStruct((B, SQ, D_MODEL), jnp.float32),
        in_specs=[pl.BlockSpec(memory_space=pltpu.VMEM)] * 5,
        out_specs=pl.BlockSpec(memory_space=pltpu.VMEM),
        scratch_shapes=[
            pltpu.VMEM((B, SQ, D_MODEL), jnp.bfloat16),
            pltpu.VMEM((3, 3, 3, SQ, PCOL), jnp.bfloat16),
            pltpu.SemaphoreType.DMA((3, 3, 3)),
            pltpu.SemaphoreType.DMA((3, 3, 3)),
        ],
        compiler_params=pltpu.CompilerParams(
            vmem_limit_bytes=100 * 1024 * 1024,
            collective_id=0,
        ),
    )(x, Wq, K_ext, V_ext, Wo)


# device time: 57291 ns/iter; 1.0373x vs baseline; 1.0373x over previous
import jax
import jax.numpy as jnp
from jax import lax
from jax.experimental import pallas as pl
from jax.experimental.pallas import tpu as pltpu

N_DEV = 8
B = 2
SQ = 512
SKV = 512
H_PER = 8
DH = 64
D_MODEL = 768
D_HID = H_PER * DH

PCOL = D_MODEL // 3

PIECES = ((0, 0, SQ), (1, 0, SQ // 2), (1, SQ // 2, SQ // 2))

ORDER = ((2, 1, 0), (1, 0, 2), (0, 2, 1))


def kernel(x, Wq, K_ext, V_ext, Wo):
    def body(x_ref, wq_ref, k_ref, v_ref, wo_ref, out_ref,
             send_ref, recv_ref, send_sems, recv_sems):
        my = lax.axis_index("i")
        lab = my ^ ((my >> 1) & 1)

        def pos_of(l):
            return l ^ ((l >> 1) & 1)

        qi = lax.broadcasted_iota(jnp.int32, (SQ, SKV), 0)
        ki = lax.broadcasted_iota(jnp.int32, (SQ, SKV), 1)
        mask = (jnp.abs(qi - ki) <= 128) | (ki < 32) | (qi < 32)
        neg = jnp.float32(-1e9)
        bf16 = jnp.bfloat16

        wq_s = wq_ref[:, pl.ds(my * D_HID, D_HID)] * 0.125

        def project_q(b):
            return jnp.dot(x_ref[b], wq_s,
                           preferred_element_type=jnp.float32)

        def attn_rows(qb, b, r0, nr):
            acc = jnp.zeros((nr, D_MODEL), jnp.float32)
            for h in range(H_PER):
                qh = qb[r0:r0 + nr, h * DH:(h + 1) * DH]
                kh = k_ref[b, :, h, :]
                s = lax.dot_general(
                    qh, kh, (((1,), (1,)), ((), ())),
                    preferred_element_type=jnp.float32)
                e = jnp.exp(jnp.where(mask[r0:r0 + nr, :], s, neg))
                ctx = jnp.dot(e, v_ref[b, :, h, :],
                              preferred_element_type=jnp.float32)
                ctx = ctx / jnp.sum(e, axis=1, keepdims=True)
                acc = acc + jnp.dot(
                    ctx, wo_ref[pl.ds(my * D_HID + h * DH, DH), :],
                    preferred_element_type=jnp.float32)
            send_ref[b, r0:r0 + nr, :] = acc.astype(bf16)

        def start_round(p, r):
            b, r0, nr = PIECES[p]
            inflight = []
            for j in range(3):
                bit = ORDER[j][r]
                partner = pos_of(lab ^ (1 << bit))
                rdma = pltpu.make_async_remote_copy(
                    src_ref=send_ref.at[b, pl.ds(r0, nr),
                                        pl.ds(j * PCOL, PCOL)],
                    dst_ref=recv_ref.at[p, r, j, pl.ds(0, nr)],
                    send_sem=send_sems.at[p, r, j],
                    recv_sem=recv_sems.at[p, r, j],
                    device_id=(partner,),
                    device_id_type=pl.DeviceIdType.MESH,
                )
                rdma.start()
                inflight.append(rdma)
            return inflight

        def finish_round(p, r, inflight):
            b, r0, nr = PIECES[p]
            for rdma in inflight:
                rdma.wait()
            for j in range(3):
                rows = pl.ds(r0, nr)
                cols = pl.ds(j * PCOL, PCOL)
                send_ref[b, rows, cols] = (
                    send_ref[b, rows, cols] + recv_ref[p, r, j, :nr])
            if r == 2:
                out_ref[b, r0:r0 + nr, :] = (
                    send_ref[b, r0:r0 + nr, :].astype(jnp.float32))

        barrier = pltpu.get_barrier_semaphore()
        for bit in range(3):
            pl.semaphore_signal(
                barrier, inc=1,
                device_id=(pos_of(lab ^ (1 << bit)),),
                device_id_type=pl.DeviceIdType.MESH,
            )

        qb0 = project_q(0)
        attn_rows(qb0, 0, 0, SQ)
        pl.semaphore_wait(barrier, 3)
        fly_a = start_round(0, 0)
        qb1 = project_q(1)
        attn_rows(qb1, 1, 0, SQ // 2)
        fly_b = start_round(1, 0)
        finish_round(0, 0, fly_a)
        fly_a = start_round(0, 1)
        attn_rows(qb1, 1, SQ // 2, SQ // 2)
        fly_c = start_round(2, 0)
        finish_round(0, 1, fly_a)
        fly_a = start_round(0, 2)
        finish_round(1, 0, fly_b)
        fly_b = start_round(1, 1)
        finish_round(0, 2, fly_a)
        finish_round(2, 0, fly_c)
        fly_c = start_round(2, 1)
        finish_round(1, 1, fly_b)
        fly_b = start_round(1, 2)
        finish_round(2, 1, fly_c)
        fly_c = start_round(2, 2)
        finish_round(1, 2, fly_b)
        finish_round(2, 2, fly_c)

    return pl.pallas_call(
        body,
        out_shape=jax.ShapeDtypeStruct((B, SQ, D_MODEL), jnp.float32),
        in_specs=[pl.BlockSpec(memory_space=pltpu.VMEM)] * 5,
        out_specs=pl.BlockSpec(memory_space=pltpu.VMEM),
        scratch_shapes=[
            pltpu.VMEM((B, SQ, D_MODEL), jnp.bfloat16),
            pltpu.VMEM((3, 3, 3, SQ, PCOL), jnp.bfloat16),
            pltpu.SemaphoreType.DMA((3, 3, 3)),
            pltpu.SemaphoreType.DMA((3, 3, 3)),
        ],
        compiler_params=pltpu.CompilerParams(
            vmem_limit_bytes=100 * 1024 * 1024,
            collective_id=0,
        ),
    )(x, Wq, K_ext, V_ext, Wo)


# device time: 56624 ns/iter; 1.0495x vs baseline; 1.0118x over previous
import jax
import jax.numpy as jnp
from jax import lax
from jax.experimental import pallas as pl
from jax.experimental.pallas import tpu as pltpu

N_DEV = 8
B = 2
SQ = 512
SKV = 512
H_PER = 8
DH = 64
D_MODEL = 768
D_HID = H_PER * DH

PCOL = D_MODEL // 3

ORDER = ((2, 1, 0), (1, 0, 2), (0, 2, 1))


def kernel(x, Wq, K_ext, V_ext, Wo):
    def body(x_ref, wq_ref, k_ref, v_ref, wo_ref, out_ref,
             send_ref, recv_ref, send_sems, recv_sems):
        my = lax.axis_index("i")
        lab = my ^ ((my >> 1) & 1)

        def pos_of(l):
            return l ^ ((l >> 1) & 1)

        qi = lax.broadcasted_iota(jnp.int32, (SQ, SKV), 0)
        ki = lax.broadcasted_iota(jnp.int32, (SQ, SKV), 1)
        mask = (jnp.abs(qi - ki) <= 128) | (ki < 32) | (qi < 32)
        neg = jnp.float32(-1e9)

        wq_s = wq_ref[:, pl.ds(my * D_HID, D_HID)] * 0.125

        def compute_half(b):
            qb = jnp.dot(x_ref[b], wq_s,
                         preferred_element_type=jnp.float32)
            acc = jnp.zeros((SQ, D_MODEL), jnp.float32)
            for h in range(H_PER):
                qh = qb[:, h * DH:(h + 1) * DH]
                kh = k_ref[b, :, h, :]
                s = lax.dot_general(
                    qh, kh, (((1,), (1,)), ((), ())),
                    preferred_element_type=jnp.float32)
                e = jnp.exp(jnp.where(mask, s, neg))
                ctx = jnp.dot(e, v_ref[b, :, h, :],
                              preferred_element_type=jnp.float32)
                ctx = ctx / jnp.sum(e, axis=1, keepdims=True)
                acc = acc + jnp.dot(
                    ctx, wo_ref[pl.ds(my * D_HID + h * DH, DH), :],
                    preferred_element_type=jnp.float32)
            send_ref[b] = acc.astype(jnp.bfloat16)

        def start_round(b, r):
            inflight = []
            for j in range(3):
                bit = ORDER[j][r]
                partner = pos_of(lab ^ (1 << bit))
                rdma = pltpu.make_async_remote_copy(
                    src_ref=send_ref.at[b, :, pl.ds(j * PCOL, PCOL)],
                    dst_ref=recv_ref.at[b, r, j],
                    send_sem=send_sems.at[b, r, j],
                    recv_sem=recv_sems.at[b, r, j],
                    device_id=(partner,),
                    device_id_type=pl.DeviceIdType.MESH,
                )
                rdma.start()
                inflight.append(rdma)
            return inflight

        def finish_round(b, r, inflight):
            for rdma in inflight:
                rdma.wait()
            for j in range(3):
                cols = pl.ds(j * PCOL, PCOL)
                send_ref[b, :, cols] = (
                    send_ref[b, :, cols] + recv_ref[b, r, j])
            if r == 2:
                out_ref[b] = send_ref[b].astype(jnp.float32)

        barrier = pltpu.get_barrier_semaphore()
        for bit in range(3):
            pl.semaphore_signal(
                barrier, inc=1,
                device_id=(pos_of(lab ^ (1 << bit)),),
                device_id_type=pl.DeviceIdType.MESH,
            )

        compute_half(0)
        pl.semaphore_wait(barrier, 3)
        fly0 = start_round(0, 0)
        compute_half(1)
        fly1 = start_round(1, 0)
        for r in range(3):
            finish_round(0, r, fly0)
            if r < 2:
                fly0 = start_round(0, r + 1)
            finish_round(1, r, fly1)
            if r < 2:
                fly1 = start_round(1, r + 1)

    return pl.pallas_call(
        body,
        out_shape=jax.ShapeDtypeStruct((B, SQ, D_MODEL), jnp.float32),
        in_specs=[pl.BlockSpec(memory_space=pltpu.VMEM)] * 5,
        out_specs=pl.BlockSpec(memory_space=pltpu.VMEM),
        scratch_shapes=[
            pltpu.VMEM((B, SQ, D_MODEL), jnp.bfloat16),
            pltpu.VMEM((B, 3, 3, SQ, PCOL), jnp.bfloat16),
            pltpu.SemaphoreType.DMA((B, 3, 3)),
            pltpu.SemaphoreType.DMA((B, 3, 3)),
        ],
        compiler_params=pltpu.CompilerParams(
            vmem_limit_bytes=100 * 1024 * 1024,
            collective_id=0,
        ),
    )(x, Wq, K_ext, V_ext, Wo)
